# baseline (speedup 1.0000x reference)
import functools

import jax
import jax.numpy as jnp
import numpy as np
from jax import lax
from jax.experimental import pallas as pl
from jax.experimental.pallas import tpu as pltpu
from jax.experimental.pallas import tpu_sc as plsc

_V = 100000
_S = 2048
_D = 1024
_B = 4

_NC, _NS = 2, 16
_NW = _NC * _NS
_POS_PER_W = _S // _NW
_CHUNK = 32
_NPC = _POS_PER_W // _CHUNK
_LANES = 16
_VECS_PER_ROW = _D // _LANES


def _positional_encoding(seq: int, d: int) -> jnp.ndarray:
    pos = np.arange(seq, dtype=np.float32)[:, None]
    i = np.arange(d, dtype=np.float32)[None, :]
    ang = pos / np.power(10000.0, (2.0 * np.floor(i / 2.0)) / float(d))
    pe = np.zeros((seq, d), dtype=np.float32)
    pe[:, 0::2] = np.sin(ang[:, 0::2])
    pe[:, 1::2] = np.cos(ang[:, 1::2])
    return jnp.asarray(pe)


def _add_pe(rows_v, pe_v):
    @plsc.parallel_loop(0, _CHUNK, 1)
    def _(r):
        for c in range(0, _D, _LANES):
            plsc.addupdate(
                rows_v.at[r, pl.ds(c, _LANES)],
                pe_v[r, pl.ds(c, _LANES)],
            )


def _body(x_hbm, table_hbm, pe_hbm, out_hbm,
          idx_v, pe_v, rows_a, rows_b,
          g_sem_a, g_sem_b, s_sem_a, s_sem_b):
    wid = lax.axis_index("s") * _NC + lax.axis_index("c")
    pos0 = wid * _POS_PER_W

    pltpu.sync_copy(x_hbm.at[pl.ds(wid * _B * _POS_PER_W, _B * _POS_PER_W)],
                    idx_v)

    units = [(pc, b) for pc in range(_NPC) for b in range(_B)]
    row_bufs = (rows_a, rows_b)
    g_sems = (g_sem_a, g_sem_b)
    s_sems = (s_sem_a, s_sem_b)

    def gather(i):
        k = i % 2
        idx = idx_v.at[pl.ds(i * _CHUNK, _CHUNK)]
        return pltpu.async_copy(table_hbm.at[idx], row_bufs[k], g_sems[k])

    gathers = {0: gather(0)}
    stores = {}
    for i in range(len(units)):
        pc, b = units[i]
        k = i % 2
        if i % _B == 0:
            pltpu.sync_copy(pe_hbm.at[pl.ds(pos0 + pc * _CHUNK, _CHUNK)], pe_v)
        gathers[i].wait()
        _add_pe(row_bufs[k], pe_v)
        if i - 1 in stores:
            stores[i - 1].wait()
        flat = b * _S + pos0 + pc * _CHUNK
        stores[i] = pltpu.async_copy(
            row_bufs[k], out_hbm.at[pl.ds(flat, _CHUNK)], s_sems[k])
        if i + 1 < len(units):
            gathers[i + 1] = gather(i + 1)
    stores[len(units) - 1].wait()


@jax.jit
def _run(x2d, table, pe):
    mesh = plsc.VectorSubcoreMesh(
        core_axis_name="c", subcore_axis_name="s",
        num_cores=_NC, num_subcores=_NS,
    )
    f = pl.kernel(
        _body,
        out_type=jax.ShapeDtypeStruct((_B * _S, _D), jnp.float32),
        mesh=mesh,
        scratch_types=[
            pltpu.VMEM((_B * _POS_PER_W,), jnp.int32),
            pltpu.VMEM((_CHUNK, _D), jnp.float32),
            pltpu.VMEM((_CHUNK, _D), jnp.float32),
            pltpu.VMEM((_CHUNK, _D), jnp.float32),
            pltpu.SemaphoreType.DMA,
            pltpu.SemaphoreType.DMA,
            pltpu.SemaphoreType.DMA,
            pltpu.SemaphoreType.DMA,
        ],
    )
    return f(x2d, table, pe)


def kernel(x, table):
    pe = _positional_encoding(_S, _D)
    x_t = (x.astype(jnp.int32)
           .reshape(_B, _NW, _NPC, _CHUNK)
           .transpose(1, 2, 0, 3)
           .reshape(-1))
    out = _run(x_t, table, pe)
    return out.reshape(_B, _S, _D)


# speedup vs baseline: 1.0192x; 1.0192x over previous
import functools

import jax
import jax.numpy as jnp
import numpy as np
from jax import lax
from jax.experimental import pallas as pl
from jax.experimental.pallas import tpu as pltpu
from jax.experimental.pallas import tpu_sc as plsc

_V = 100000
_S = 2048
_D = 1024
_B = 4

_NC, _NS = 2, 16
_NW = _NC * _NS
_POS_PER_W = _S // _NW
_PC = 8
_NSTEPS = _POS_PER_W // _PC
_ROWS = _PC * _B
_LANES = 16


def _positional_encoding(seq: int, d: int) -> jnp.ndarray:
    pos = np.arange(seq, dtype=np.float32)[:, None]
    i = np.arange(d, dtype=np.float32)[None, :]
    ang = pos / np.power(10000.0, (2.0 * np.floor(i / 2.0)) / float(d))
    pe = np.zeros((seq, d), dtype=np.float32)
    pe[:, 0::2] = np.sin(ang[:, 0::2])
    pe[:, 1::2] = np.cos(ang[:, 1::2])
    return jnp.asarray(pe)


def _add_pe(rows_v, pe_v):
    @plsc.parallel_loop(0, _PC, 1)
    def _(r):
        for c in range(0, _D, _LANES):
            v = pe_v[r, pl.ds(c, _LANES)]
            for b in range(_B):
                plsc.addupdate(rows_v.at[b * _PC + r, pl.ds(c, _LANES)], v)


def _body(x_hbm, table_hbm, pe_hbm, out_hbm,
          idx_v, pe_a, pe_b, rows_a, rows_b,
          g_sem_a, g_sem_b, p_sem_a, p_sem_b, s_sem_a, s_sem_b):
    wid = lax.axis_index("s") * _NC + lax.axis_index("c")
    pos0 = wid * _POS_PER_W

    pltpu.sync_copy(x_hbm.at[pl.ds(wid * _B * _POS_PER_W, _B * _POS_PER_W)],
                    idx_v)

    row_bufs = (rows_a, rows_b)
    pe_bufs = (pe_a, pe_b)
    g_sems = (g_sem_a, g_sem_b)
    p_sems = (p_sem_a, p_sem_b)
    s_sems = (s_sem_a, s_sem_b)

    def start_step(s):
        k = s % 2
        idx = idx_v.at[pl.ds(s * _ROWS, _ROWS)]
        g = pltpu.async_copy(table_hbm.at[idx], row_bufs[k], g_sems[k])
        p = pltpu.async_copy(pe_hbm.at[pl.ds(pos0 + s * _PC, _PC)],
                             pe_bufs[k], p_sems[k])
        return g, p

    inflight = {0: start_step(0)}
    stores = {}
    for s in range(_NSTEPS):
        k = s % 2
        g, p = inflight[s]
        g.wait()
        p.wait()
        _add_pe(row_bufs[k], pe_bufs[k])
        if s - 1 in stores:
            for st in stores[s - 1]:
                st.wait()
        stores[s] = []
        for b in range(_B):
            flat = b * _S + pos0 + s * _PC
            stores[s].append(pltpu.async_copy(
                row_bufs[k].at[pl.ds(b * _PC, _PC)],
                out_hbm.at[pl.ds(flat, _PC)], s_sems[k]))
        if s + 1 < _NSTEPS:
            inflight[s + 1] = start_step(s + 1)
    for st in stores[_NSTEPS - 1]:
        st.wait()


@jax.jit
def _run(x_t, table, pe):
    mesh = plsc.VectorSubcoreMesh(
        core_axis_name="c", subcore_axis_name="s",
        num_cores=_NC, num_subcores=_NS,
    )
    f = pl.kernel(
        _body,
        out_type=jax.ShapeDtypeStruct((_B * _S, _D), jnp.float32),
        mesh=mesh,
        scratch_types=[
            pltpu.VMEM((_B * _POS_PER_W,), jnp.int32),
            pltpu.VMEM((_PC, _D), jnp.float32),
            pltpu.VMEM((_PC, _D), jnp.float32),
            pltpu.VMEM((_ROWS, _D), jnp.float32),
            pltpu.VMEM((_ROWS, _D), jnp.float32),
            pltpu.SemaphoreType.DMA,
            pltpu.SemaphoreType.DMA,
            pltpu.SemaphoreType.DMA,
            pltpu.SemaphoreType.DMA,
            pltpu.SemaphoreType.DMA,
            pltpu.SemaphoreType.DMA,
        ],
    )
    return f(x_t, table, pe)


def kernel(x, table):
    pe = _positional_encoding(_S, _D)
    x_t = (x.astype(jnp.int32)
           .reshape(_B, _NW, _NSTEPS, _PC)
           .transpose(1, 2, 0, 3)
           .reshape(-1))
    out = _run(x_t, table, pe)
    return out.reshape(_B, _S, _D)


# speedup vs baseline: 1.2197x; 1.1967x over previous
import functools

import jax
import jax.numpy as jnp
import numpy as np
from jax import lax
from jax.experimental import pallas as pl
from jax.experimental.pallas import tpu as pltpu
from jax.experimental.pallas import tpu_sc as plsc

_V = 100000
_S = 2048
_D = 1024
_B = 4

_NC, _NS = 2, 16
_NW = _NC * _NS
_POS_PER_W = _S // _NW
_PC = 8
_NSTEPS = _POS_PER_W // _PC
_ROWS = _PC * _B
_LANES = 16


def _positional_encoding(seq: int, d: int) -> jnp.ndarray:
    pos = np.arange(seq, dtype=np.float32)[:, None]
    i = np.arange(d, dtype=np.float32)[None, :]
    ang = pos / np.power(10000.0, (2.0 * np.floor(i / 2.0)) / float(d))
    pe = np.zeros((seq, d), dtype=np.float32)
    pe[:, 0::2] = np.sin(ang[:, 0::2])
    pe[:, 1::2] = np.cos(ang[:, 1::2])
    return jnp.asarray(pe)


def _add_pe(rows_v, pe_v):
    @plsc.parallel_loop(0, _PC, 1)
    def _(r):
        for c in range(0, _D, _LANES):
            v = pe_v[r, pl.ds(c, _LANES)]
            for b in range(_B):
                plsc.addupdate(rows_v.at[b * _PC + r, pl.ds(c, _LANES)], v)


_NBUF = 3


def _body(x_hbm, table_hbm, pe_hbm, out_hbm,
          idx_v, pe_0, pe_1, pe_2, rows_0, rows_1, rows_2,
          g_sem_0, g_sem_1, g_sem_2, p_sem_0, p_sem_1, p_sem_2,
          s_sem_0, s_sem_1, s_sem_2):
    wid = lax.axis_index("s") * _NC + lax.axis_index("c")
    pos0 = wid * _POS_PER_W

    pltpu.sync_copy(x_hbm.at[pl.ds(wid * _B * _POS_PER_W, _B * _POS_PER_W)],
                    idx_v)

    row_bufs = (rows_0, rows_1, rows_2)
    pe_bufs = (pe_0, pe_1, pe_2)
    g_sems = (g_sem_0, g_sem_1, g_sem_2)
    p_sems = (p_sem_0, p_sem_1, p_sem_2)
    s_sems = (s_sem_0, s_sem_1, s_sem_2)

    def start_step(s):
        k = s % _NBUF
        idx = idx_v.at[pl.ds(s * _ROWS, _ROWS)]
        g = pltpu.async_copy(table_hbm.at[idx], row_bufs[k], g_sems[k])
        p = pltpu.async_copy(pe_hbm.at[pl.ds(pos0 + s * _PC, _PC)],
                             pe_bufs[k], p_sems[k])
        return g, p

    inflight = {s: start_step(s) for s in range(min(_NBUF - 1, _NSTEPS))}
    stores = {}
    for s in range(_NSTEPS):
        k = s % _NBUF
        g, p = inflight[s]
        g.wait()
        p.wait()
        _add_pe(row_bufs[k], pe_bufs[k])
        if s - 1 in stores:
            for st in stores[s - 1]:
                st.wait()
        stores[s] = []
        for b in range(_B):
            flat = b * _S + pos0 + s * _PC
            stores[s].append(pltpu.async_copy(
                row_bufs[k].at[pl.ds(b * _PC, _PC)],
                out_hbm.at[pl.ds(flat, _PC)], s_sems[k]))
        if s + _NBUF - 1 < _NSTEPS:
            inflight[s + _NBUF - 1] = start_step(s + _NBUF - 1)
    for st in stores[_NSTEPS - 1]:
        st.wait()


@jax.jit
def _run(x_t, table, pe):
    mesh = plsc.VectorSubcoreMesh(
        core_axis_name="c", subcore_axis_name="s",
        num_cores=_NC, num_subcores=_NS,
    )
    f = pl.kernel(
        _body,
        out_type=jax.ShapeDtypeStruct((_B * _S, _D), jnp.float32),
        mesh=mesh,
        scratch_types=(
            [pltpu.VMEM((_B * _POS_PER_W,), jnp.int32)]
            + [pltpu.VMEM((_PC, _D), jnp.float32)] * _NBUF
            + [pltpu.VMEM((_ROWS, _D), jnp.float32)] * _NBUF
            + [pltpu.SemaphoreType.DMA] * (3 * _NBUF)
        ),
    )
    return f(x_t, table, pe)


def kernel(x, table):
    pe = _positional_encoding(_S, _D)
    x_t = (x.astype(jnp.int32)
           .reshape(_B, _NW, _NSTEPS, _PC)
           .transpose(1, 2, 0, 3)
           .reshape(-1))
    out = _run(x_t, table, pe)
    return out.reshape(_B, _S, _D)
